# baseline (speedup 1.0000x reference)
import functools
import jax
import jax.numpy as jnp
from jax.experimental import pallas as pl
from jax.experimental.pallas import tpu as pltpu

NX = 512
NY = 512
VS = 0.2
X0 = -51.2
NSEG = NX * NY
FEAT = 64
NB = 2048
NPTS = 100000
NPAD = 102400


def _voxel_f(p):
    return jnp.clip(jnp.floor((p - X0) / VS), 0.0, 511.0)


def _x_linear(pts, fea, gath, W1, W2, W3, b):
    cnt = jnp.maximum(gath[:, 0:1], 1.0)
    mean = gath[:, 1:4] / cnt
    fcl = pts - mean
    px = pts[:, 0:1]
    py = pts[:, 1:2]
    vxf = _voxel_f(px)
    vyf = _voxel_f(py)
    cx = (vxf + 0.5) * VS + X0
    cy = (vyf + 0.5) * VS + X0
    fcen = jnp.concatenate([px - cx, py - cy], axis=1)
    x = fea @ W1 + fcl @ W2 + fcen @ W3 + b
    return x


def _valid_mask(i):
    row = jax.lax.broadcasted_iota(jnp.int32, (NB, 1), 0) + i * NB
    return row < NPTS


def _stats_kernel(pts_ref, fea_ref, gath_ref, W1_ref, W2_ref, W3_ref, b_ref,
                  out_ref):
    i = pl.program_id(1)
    x = _x_linear(pts_ref[0], fea_ref[0], gath_ref[0], W1_ref[...],
                  W2_ref[...], W3_ref[...], b_ref[...])
    x = jnp.where(_valid_mask(i), x, 0.0)
    blk = jnp.concatenate([jnp.sum(x, 0, keepdims=True),
                           jnp.sum(x * x, 0, keepdims=True)], axis=0)

    @pl.when(i == 0)
    def _():
        out_ref[0] = blk

    @pl.when(i != 0)
    def _():
        out_ref[0] += blk


def _norm_kernel(pts_ref, fea_ref, gath_ref, W1_ref, W2_ref, W3_ref, b_ref,
                 stats_ref, gamma_ref, beta_ref, pf_ref, scaledT_ref):
    i = pl.program_id(1)
    x = _x_linear(pts_ref[0], fea_ref[0], gath_ref[0], W1_ref[...],
                  W2_ref[...], W3_ref[...], b_ref[...])
    n = jnp.float32(NPTS)
    mu = stats_ref[0, 0:1, :] / n
    var = stats_ref[0, 1:2, :] / n - mu * mu
    scale = gamma_ref[...] * jax.lax.rsqrt(var + 1e-5)
    pf = jnp.maximum(x * scale + (beta_ref[...] - mu * scale), 0.0)
    pf_ref[0] = pf
    inv_cnt = 1.0 / jnp.maximum(gath_ref[0][:, 0:1], 1.0)
    scaled = jnp.where(_valid_mask(i), pf * inv_cnt, 0.0)
    scaledT_ref[0] = scaled.T


def _tc_passes(points, points_in_fea, gath, W, b, gamma, beta):
    B, N = points.shape[0], points.shape[1]
    nblk = N // NB
    W1 = W[:FEAT]
    W2 = W[FEAT:FEAT + 3]
    W3 = W[FEAT + 3:FEAT + 5]
    b2 = b[None, :]
    g2 = gamma[None, :]
    be2 = beta[None, :]
    grid = (B, nblk)
    in_specs = [
        pl.BlockSpec((1, NB, 3), lambda bi, i: (bi, i, 0)),
        pl.BlockSpec((1, NB, FEAT), lambda bi, i: (bi, i, 0)),
        pl.BlockSpec((1, NB, 4), lambda bi, i: (bi, i, 0)),
        pl.BlockSpec((FEAT, FEAT), lambda bi, i: (0, 0)),
        pl.BlockSpec((3, FEAT), lambda bi, i: (0, 0)),
        pl.BlockSpec((2, FEAT), lambda bi, i: (0, 0)),
        pl.BlockSpec((1, FEAT), lambda bi, i: (0, 0)),
    ]
    stats = pl.pallas_call(
        _stats_kernel,
        grid=grid,
        in_specs=in_specs,
        out_specs=pl.BlockSpec((1, 2, FEAT), lambda bi, i: (bi, 0, 0)),
        out_shape=jax.ShapeDtypeStruct((B, 2, FEAT), jnp.float32),
    )(points, points_in_fea, gath, W1, W2, W3, b2)

    pf, scaledT = pl.pallas_call(
        _norm_kernel,
        grid=grid,
        in_specs=in_specs + [
            pl.BlockSpec((1, 2, FEAT), lambda bi, i: (bi, 0, 0)),
            pl.BlockSpec((1, FEAT), lambda bi, i: (0, 0)),
            pl.BlockSpec((1, FEAT), lambda bi, i: (0, 0)),
        ],
        out_specs=[
            pl.BlockSpec((1, NB, FEAT), lambda bi, i: (bi, i, 0)),
            pl.BlockSpec((1, FEAT, NB), lambda bi, i: (bi, 0, i)),
        ],
        out_shape=[
            jax.ShapeDtypeStruct((B, N, FEAT), jnp.float32),
            jax.ShapeDtypeStruct((B, FEAT, N), jnp.float32),
        ],
    )(points, points_in_fea, gath, W1, W2, W3, b2, stats, g2, be2)
    return pf, scaledT


def kernel(points, points_in_fea, W, b, gamma, beta):
    B, N = points.shape[0], points.shape[1]
    vx = jnp.clip(jnp.floor((points[..., 0] - X0) / VS).astype(jnp.int32), 0, NX - 1)
    vy = jnp.clip(jnp.floor((points[..., 1] - X0) / VS).astype(jnp.int32), 0, NY - 1)
    seg = vy * NX + vx

    def seg_sums(seg_b, pts_b):
        ones = jnp.ones((N, 1), jnp.float32)
        rows = jnp.concatenate([ones, pts_b], axis=1)
        tab = jax.ops.segment_sum(rows, seg_b, num_segments=NSEG)
        return tab[seg_b]

    gath = jax.vmap(seg_sums)(seg, points)

    pad = [(0, 0), (0, NPAD - N), (0, 0)]
    pf, scaledT = _tc_passes(jnp.pad(points, pad), jnp.pad(points_in_fea, pad),
                             jnp.pad(gath, pad), W, b, gamma, beta)
    pf = pf[:, :N]

    def seg_scatter(seg_b, scaledT_b):
        vox = jax.ops.segment_sum(scaledT_b.T[:N], seg_b, num_segments=NSEG)
        return vox.T.reshape(FEAT, NY, NX)

    pseudo = jax.vmap(seg_scatter)(seg, scaledT)
    return pseudo, pf
